# Initial kernel scaffold; baseline (speedup 1.0000x reference)
import jax
import jax.numpy as jnp
from jax.experimental import pallas as pl


def kernel(x, edge_index, batch, W1, b1, W2, b2, W3, b3, Wc1, bc1, Wc2, bc2):
    raise NotImplementedError("write your pallas kernel here")


# speedup vs baseline: 12.0567x; 12.0567x over previous
import functools

import jax
import jax.numpy as jnp
from jax import lax
from jax.experimental import pallas as pl
from jax.experimental.pallas import tpu as pltpu
from jax.experimental.pallas import tpu_sc as plsc

N = 10000
E = 320000
D_IN = 128
H = 64
G = 16
OUT = 2

NC = 2
NS = 16
NW = NC * NS

C = 128
CHUNKS = 79
EPW = C * CHUNKS
E_PAD = NW * EPW
ROWS = 10240
RPT = ROWS // NS
OPT = 624
TAIL = N - NS * OPT
DW = 16


@functools.cache
def _sc_kernels():
    mesh = plsc.VectorSubcoreMesh(core_axis_name="c", subcore_axis_name="s",
                                  num_cores=NC, num_subcores=NS)

    @functools.partial(
        pl.kernel,
        out_type=jax.ShapeDtypeStruct((NC, N, DW), jnp.float32),
        mesh=mesh,
        scratch_types=[
            pltpu.VMEM((C,), jnp.int32),
            pltpu.VMEM((C, DW), jnp.float32),
            pltpu.VMEM_SHARED((ROWS, DW), jnp.float32),
        ],
        compiler_params=pltpu.CompilerParams(use_tc_tiling_on_sc=False),
    )
    def sc_deg(dst_hbm, ones_hbm, zeros_hbm, out_hbm, dst_v, ones_v, acc):
        cid = lax.axis_index("c")
        sid = lax.axis_index("s")
        wid = sid * NC + cid
        pltpu.sync_copy(zeros_hbm, acc.at[pl.ds(sid * RPT, RPT)])
        pltpu.sync_copy(ones_hbm, ones_v)
        plsc.subcore_barrier()

        def body(i, carry):
            base = wid * EPW + i * C
            pltpu.sync_copy(dst_hbm.at[pl.ds(base, C)], dst_v)
            pltpu.sync_copy(ones_v, acc.at[dst_v], add=True)
            return carry

        lax.fori_loop(0, CHUNKS, body, 0)
        plsc.subcore_barrier()
        pltpu.sync_copy(acc.at[pl.ds(sid * OPT, OPT)],
                        out_hbm.at[cid, pl.ds(sid * OPT, OPT)])

        @pl.when(sid == NS - 1)
        def _():
            pltpu.sync_copy(acc.at[pl.ds(NS * OPT, TAIL)],
                            out_hbm.at[cid, pl.ds(NS * OPT, TAIL)])

    @functools.partial(
        pl.kernel,
        out_type=jax.ShapeDtypeStruct((NC, N, H), jnp.float32),
        mesh=mesh,
        scratch_types=[
            pltpu.VMEM((C,), jnp.int32),
            pltpu.VMEM((C,), jnp.int32),
            pltpu.VMEM((C, H), jnp.float32),
            pltpu.VMEM_SHARED((ROWS, H), jnp.float32),
            pltpu.SemaphoreType.DMA,
        ],
        compiler_params=pltpu.CompilerParams(use_tc_tiling_on_sc=False),
    )
    def sc_edge(u_hbm, src_hbm, dst_hbm, zeros_hbm, out_hbm,
                src_v, dst_v, rows_v, acc, sem):
        cid = lax.axis_index("c")
        sid = lax.axis_index("s")
        wid = sid * NC + cid
        pltpu.sync_copy(zeros_hbm, acc.at[pl.ds(sid * RPT, RPT)])
        plsc.subcore_barrier()

        def body(i, carry):
            base = wid * EPW + i * C
            pltpu.sync_copy(src_hbm.at[pl.ds(base, C)], src_v)
            pltpu.sync_copy(dst_hbm.at[pl.ds(base, C)], dst_v)
            pltpu.async_copy(u_hbm.at[src_v], rows_v, sem).wait()
            pltpu.sync_copy(rows_v, acc.at[dst_v], add=True)
            return carry

        lax.fori_loop(0, CHUNKS, body, 0)
        plsc.subcore_barrier()
        pltpu.sync_copy(acc.at[pl.ds(sid * OPT, OPT)],
                        out_hbm.at[cid, pl.ds(sid * OPT, OPT)])

        @pl.when(sid == NS - 1)
        def _():
            pltpu.sync_copy(acc.at[pl.ds(NS * OPT, TAIL)],
                            out_hbm.at[cid, pl.ds(NS * OPT, TAIL)])

    return sc_deg, sc_edge


BLK = 1000


def _tc1_body(x_ref, w_ref, degp_ref, u_ref, dinv_ref):
    d = degp_ref[...]
    deg = d[0, :, :1] + d[1, :, :1] + 1.0
    dinv = lax.rsqrt(deg)
    h = jnp.dot(x_ref[...], w_ref[...], preferred_element_type=jnp.float32)
    u_ref[...] = h * dinv
    dinv_ref[...] = jnp.broadcast_to(dinv, (BLK, H))


_tc1 = pl.pallas_call(
    _tc1_body,
    grid=(N // BLK,),
    in_specs=[
        pl.BlockSpec((BLK, D_IN), lambda i: (i, 0)),
        pl.BlockSpec((D_IN, H), lambda i: (0, 0)),
        pl.BlockSpec((NC, BLK, DW), lambda i: (0, i, 0)),
    ],
    out_specs=[
        pl.BlockSpec((BLK, H), lambda i: (i, 0)),
        pl.BlockSpec((BLK, H), lambda i: (i, 0)),
    ],
    out_shape=[
        jax.ShapeDtypeStruct((N, H), jnp.float32),
        jax.ShapeDtypeStruct((N, H), jnp.float32),
    ],
)


def _tc_mid_body(sp_ref, u_ref, dinv_ref, b_ref, w_ref, out_ref):
    sp = sp_ref[...]
    dinv = dinv_ref[...]
    h = jnp.maximum((sp[0] + sp[1] + u_ref[...]) * dinv + b_ref[...], 0.0)
    out_ref[...] = jnp.dot(
        h, w_ref[...], preferred_element_type=jnp.float32) * dinv


_tc_mid = pl.pallas_call(
    _tc_mid_body,
    grid=(N // BLK,),
    in_specs=[
        pl.BlockSpec((NC, BLK, H), lambda i: (0, i, 0)),
        pl.BlockSpec((BLK, H), lambda i: (i, 0)),
        pl.BlockSpec((BLK, H), lambda i: (i, 0)),
        pl.BlockSpec((1, H), lambda i: (0, 0)),
        pl.BlockSpec((H, H), lambda i: (0, 0)),
    ],
    out_specs=pl.BlockSpec((BLK, H), lambda i: (i, 0)),
    out_shape=jax.ShapeDtypeStruct((N, H), jnp.float32),
)


def _tc4_body(sp_ref, u_ref, dinv_ref, b_ref, batch_ref,
              wc1_ref, bc1_ref, wc2_ref, bc2_ref, out_ref):
    sp = sp_ref[...]
    h3 = (sp[0] + sp[1] + u_ref[...]) * dinv_ref[...] + b_ref[...]
    bt = batch_ref[...]
    sums, maxs, cnts = [], [], []
    neg_inf = jnp.float32(-jnp.inf)
    for g in range(G):
        m = bt == g
        sums.append(jnp.sum(jnp.where(m, h3, 0.0), axis=0, keepdims=True))
        maxs.append(jnp.max(jnp.where(m, h3, neg_inf), axis=0, keepdims=True))
        cnts.append(jnp.sum(m.astype(jnp.float32), axis=0, keepdims=True))
    seg_sum = jnp.concatenate(sums, axis=0)
    seg_max = jnp.concatenate(maxs, axis=0)
    counts = jnp.concatenate(cnts, axis=0)
    mean = seg_sum / jnp.maximum(counts, 1.0)
    feat = jnp.concatenate([mean, seg_max], axis=1)
    z = jnp.maximum(
        jnp.dot(feat, wc1_ref[...], preferred_element_type=jnp.float32)
        + bc1_ref[...], 0.0)
    out_ref[...] = jnp.dot(
        z, wc2_ref[...], preferred_element_type=jnp.float32) + bc2_ref[...]


_tc4 = pl.pallas_call(
    _tc4_body,
    out_shape=jax.ShapeDtypeStruct((G, OUT), jnp.float32),
)


def kernel(x, edge_index, batch, W1, b1, W2, b2, W3, b3, Wc1, bc1, Wc2, bc2):
    src = edge_index[0]
    dst = edge_index[1]
    pad = E_PAD - E
    src_p = jnp.concatenate([src, jnp.zeros((pad,), jnp.int32)])
    dst_p = jnp.concatenate([dst, jnp.full((pad,), N, jnp.int32)])
    zeros_h = jnp.zeros((RPT, H), jnp.float32)
    zeros_d = jnp.zeros((RPT, DW), jnp.float32)
    ones_d = jnp.ones((C, DW), jnp.float32)

    sc_deg, sc_edge = _sc_kernels()
    degp = sc_deg(dst_p, ones_d, zeros_d)
    u1, dinv = _tc1(x, W1, degp)
    s1 = sc_edge(u1, src_p, dst_p, zeros_h)
    u2 = _tc_mid(s1, u1, dinv, b1.reshape(1, H), W2)
    s2 = sc_edge(u2, src_p, dst_p, zeros_h)
    u3 = _tc_mid(s2, u2, dinv, b2.reshape(1, H), W3)
    s3 = sc_edge(u3, src_p, dst_p, zeros_h)
    logits = _tc4(s3, u3, dinv, b3.reshape(1, H), batch.reshape(N, 1),
                  Wc1, bc1.reshape(1, H), Wc2, bc2.reshape(1, OUT))
    return logits


# speedup vs baseline: 14.9367x; 1.2389x over previous
import functools

import jax
import jax.numpy as jnp
from jax import lax
from jax.experimental import pallas as pl
from jax.experimental.pallas import tpu as pltpu
from jax.experimental.pallas import tpu_sc as plsc

N = 10000
E = 320000
D_IN = 128
H = 64
G = 16
OUT = 2

NC = 2
NS = 16
NW = NC * NS

C = 128
CHUNKS = 80
EPW = C * CHUNKS
E_PAD = NW * EPW
CPG = 2
NG = CHUNKS // CPG
NQ = 4
ROWS = 10240
RPT = ROWS // NS
OPT = 624
TAIL = N - NS * OPT
DW = 16


@functools.cache
def _sc_kernels():
    mesh = plsc.VectorSubcoreMesh(core_axis_name="c", subcore_axis_name="s",
                                  num_cores=NC, num_subcores=NS)

    def copy_out(acc, out_hbm, cid, sid):
        pltpu.sync_copy(acc.at[pl.ds(sid * OPT, OPT)],
                        out_hbm.at[cid, pl.ds(sid * OPT, OPT)])

        @pl.when(sid == NS - 1)
        def _():
            pltpu.sync_copy(acc.at[pl.ds(NS * OPT, TAIL)],
                            out_hbm.at[cid, pl.ds(NS * OPT, TAIL)])

    @functools.partial(
        pl.kernel,
        out_type=jax.ShapeDtypeStruct((NC, N, DW), jnp.float32),
        mesh=mesh,
        scratch_types=[
            pltpu.VMEM((CHUNKS, C), jnp.int32),
            pltpu.VMEM((C, DW), jnp.float32),
            pltpu.VMEM_SHARED((ROWS, DW), jnp.float32),
            pltpu.SemaphoreType.DMA,
        ],
        compiler_params=pltpu.CompilerParams(use_tc_tiling_on_sc=False),
    )
    def sc_deg(dst_hbm, ones_hbm, zeros_hbm, out_hbm, dst_v, ones_v, acc, sem):
        cid = lax.axis_index("c")
        sid = lax.axis_index("s")
        wid = sid * NC + cid
        pltpu.sync_copy(zeros_hbm, acc.at[pl.ds(sid * RPT, RPT)])
        pltpu.sync_copy(ones_hbm, ones_v)
        pltpu.sync_copy(dst_hbm.at[pl.ds(wid * CHUNKS, CHUNKS)], dst_v)
        plsc.subcore_barrier()

        def body(t, carry):
            for k in range(4):
                pltpu.async_copy(ones_v, acc.at[dst_v.at[t * 4 + k]], sem,
                                 add=True)
            for k in range(4):
                pltpu.make_async_copy(
                    ones_v, acc.at[dst_v.at[t * 4 + k]], sem).wait()
            return carry

        lax.fori_loop(0, CHUNKS // 4, body, 0)
        plsc.subcore_barrier()
        copy_out(acc, out_hbm, cid, sid)

    @functools.partial(
        pl.kernel,
        out_type=jax.ShapeDtypeStruct((NC, N, H), jnp.float32),
        mesh=mesh,
        scratch_types=(
            [pltpu.VMEM((CHUNKS, C), jnp.int32),
             pltpu.VMEM((CHUNKS, C), jnp.int32)]
            + [pltpu.VMEM((C, H), jnp.float32) for _ in range(NQ * CPG)]
            + [pltpu.VMEM_SHARED((ROWS, H), jnp.float32)]
            + [pltpu.SemaphoreType.DMA for _ in range(2 * NQ)]
        ),
        compiler_params=pltpu.CompilerParams(use_tc_tiling_on_sc=False),
    )
    def sc_edge(u_hbm, src_hbm, dst_hbm, zeros_hbm, out_hbm,
                src_v, dst_v, b0, b1, b2, b3, b4, b5, b6, b7, acc,
                sg0, sg1, sg2, sg3, ss0, ss1, ss2, ss3):
        bufs = ((b0, b1), (b2, b3), (b4, b5), (b6, b7))
        sgs = (sg0, sg1, sg2, sg3)
        sss = (ss0, ss1, ss2, ss3)
        cid = lax.axis_index("c")
        sid = lax.axis_index("s")
        wid = sid * NC + cid
        pltpu.sync_copy(zeros_hbm, acc.at[pl.ds(sid * RPT, RPT)])
        pltpu.sync_copy(src_hbm.at[pl.ds(wid * CHUNKS, CHUNKS)], src_v)
        pltpu.sync_copy(dst_hbm.at[pl.ds(wid * CHUNKS, CHUNKS)], dst_v)
        plsc.subcore_barrier()

        def fire_gather(g, q):
            for k in range(CPG):
                pltpu.async_copy(u_hbm.at[src_v.at[g * CPG + k]],
                                 bufs[q][k], sgs[q])

        def drain_gather(g, q):
            for k in range(CPG):
                pltpu.make_async_copy(u_hbm.at[src_v.at[g * CPG + k]],
                                      bufs[q][k], sgs[q]).wait()

        def fire_scatter(g, q):
            for k in range(CPG):
                pltpu.async_copy(bufs[q][k], acc.at[dst_v.at[g * CPG + k]],
                                 sss[q], add=True)

        def drain_scatter(g, q):
            for k in range(CPG):
                pltpu.make_async_copy(bufs[q][k],
                                      acc.at[dst_v.at[g * CPG + k]],
                                      sss[q]).wait()

        fire_gather(0, 0)
        fire_gather(1, 1)

        def step(g, q):
            @pl.when(g >= 2)
            def _():
                drain_scatter(g - 2, (q + 2) % NQ)

            @pl.when(g + 2 < NG)
            def _():
                fire_gather(g + 2, (q + 2) % NQ)

            drain_gather(g, q)
            fire_scatter(g, q)

        def body(t, carry):
            for k in range(NQ):
                step(t * NQ + k, k)
            return carry

        lax.fori_loop(0, NG // NQ, body, 0)
        drain_scatter(NG - 2, (NG - 2) % NQ)
        drain_scatter(NG - 1, (NG - 1) % NQ)
        plsc.subcore_barrier()
        copy_out(acc, out_hbm, cid, sid)

    return sc_deg, sc_edge


BLK = 1000


def _tc1_body(x_ref, w_ref, degp_ref, u_ref, dinv_ref):
    d = degp_ref[...]
    deg = d[0, :, :1] + d[1, :, :1] + 1.0
    dinv = lax.rsqrt(deg)
    h = jnp.dot(x_ref[...], w_ref[...], preferred_element_type=jnp.float32)
    u_ref[...] = h * dinv
    dinv_ref[...] = jnp.broadcast_to(dinv, (BLK, H))


_tc1 = pl.pallas_call(
    _tc1_body,
    grid=(N // BLK,),
    in_specs=[
        pl.BlockSpec((BLK, D_IN), lambda i: (i, 0)),
        pl.BlockSpec((D_IN, H), lambda i: (0, 0)),
        pl.BlockSpec((NC, BLK, DW), lambda i: (0, i, 0)),
    ],
    out_specs=[
        pl.BlockSpec((BLK, H), lambda i: (i, 0)),
        pl.BlockSpec((BLK, H), lambda i: (i, 0)),
    ],
    out_shape=[
        jax.ShapeDtypeStruct((N, H), jnp.float32),
        jax.ShapeDtypeStruct((N, H), jnp.float32),
    ],
)


def _tc_mid_body(sp_ref, u_ref, dinv_ref, b_ref, w_ref, out_ref):
    sp = sp_ref[...]
    dinv = dinv_ref[...]
    h = jnp.maximum((sp[0] + sp[1] + u_ref[...]) * dinv + b_ref[...], 0.0)
    out_ref[...] = jnp.dot(
        h, w_ref[...], preferred_element_type=jnp.float32) * dinv


_tc_mid = pl.pallas_call(
    _tc_mid_body,
    grid=(N // BLK,),
    in_specs=[
        pl.BlockSpec((NC, BLK, H), lambda i: (0, i, 0)),
        pl.BlockSpec((BLK, H), lambda i: (i, 0)),
        pl.BlockSpec((BLK, H), lambda i: (i, 0)),
        pl.BlockSpec((1, H), lambda i: (0, 0)),
        pl.BlockSpec((H, H), lambda i: (0, 0)),
    ],
    out_specs=pl.BlockSpec((BLK, H), lambda i: (i, 0)),
    out_shape=jax.ShapeDtypeStruct((N, H), jnp.float32),
)


def _tc4_body(sp_ref, u_ref, dinv_ref, b_ref, batch_ref,
              wc1_ref, bc1_ref, wc2_ref, bc2_ref, out_ref):
    sp = sp_ref[...]
    h3 = (sp[0] + sp[1] + u_ref[...]) * dinv_ref[...] + b_ref[...]
    bt = batch_ref[...]
    sums, maxs, cnts = [], [], []
    neg_inf = jnp.float32(-jnp.inf)
    for g in range(G):
        m = bt == g
        sums.append(jnp.sum(jnp.where(m, h3, 0.0), axis=0, keepdims=True))
        maxs.append(jnp.max(jnp.where(m, h3, neg_inf), axis=0, keepdims=True))
        cnts.append(jnp.sum(m.astype(jnp.float32), axis=0, keepdims=True))
    seg_sum = jnp.concatenate(sums, axis=0)
    seg_max = jnp.concatenate(maxs, axis=0)
    counts = jnp.concatenate(cnts, axis=0)
    mean = seg_sum / jnp.maximum(counts, 1.0)
    feat = jnp.concatenate([mean, seg_max], axis=1)
    z = jnp.maximum(
        jnp.dot(feat, wc1_ref[...], preferred_element_type=jnp.float32)
        + bc1_ref[...], 0.0)
    out_ref[...] = jnp.dot(
        z, wc2_ref[...], preferred_element_type=jnp.float32) + bc2_ref[...]


_tc4 = pl.pallas_call(
    _tc4_body,
    out_shape=jax.ShapeDtypeStruct((G, OUT), jnp.float32),
)


def kernel(x, edge_index, batch, W1, b1, W2, b2, W3, b3, Wc1, bc1, Wc2, bc2):
    src = edge_index[0]
    dst = edge_index[1]
    pad = E_PAD - E
    src_p = jnp.concatenate(
        [src, jnp.zeros((pad,), jnp.int32)]).reshape(NW * CHUNKS, C)
    dst_p = jnp.concatenate(
        [dst, jnp.full((pad,), N, jnp.int32)]).reshape(NW * CHUNKS, C)
    zeros_h = jnp.zeros((RPT, H), jnp.float32)
    zeros_d = jnp.zeros((RPT, DW), jnp.float32)
    ones_d = jnp.ones((C, DW), jnp.float32)

    sc_deg, sc_edge = _sc_kernels()
    degp = sc_deg(dst_p, ones_d, zeros_d)
    u1, dinv = _tc1(x, W1, degp)
    s1 = sc_edge(u1, src_p, dst_p, zeros_h)
    u2 = _tc_mid(s1, u1, dinv, b1.reshape(1, H), W2)
    s2 = sc_edge(u2, src_p, dst_p, zeros_h)
    u3 = _tc_mid(s2, u2, dinv, b2.reshape(1, H), W3)
    s3 = sc_edge(u3, src_p, dst_p, zeros_h)
    logits = _tc4(s3, u3, dinv, b3.reshape(1, H), batch.reshape(N, 1),
                  Wc1, bc1.reshape(1, H), Wc2, bc2.reshape(1, OUT))
    return logits


# speedup vs baseline: 30.4160x; 2.0363x over previous
import functools

import jax
import jax.numpy as jnp
from jax import lax
from jax.experimental import pallas as pl
from jax.experimental.pallas import tpu as pltpu
from jax.experimental.pallas import tpu_sc as plsc

N = 10000
E = 320000
D_IN = 128
H = 64
G = 16
OUT = 2

NC = 2
NS = 16
NW = NC * NS
HH = H // NC

C = 128
CHUNKS = 80
EPW = C * CHUNKS
E_PAD = NW * EPW
CH2 = 160
NG = CH2 // 2
CPG = 2
NQ = 4

ROWS = 10240
RPT = ROWS // NS
ROWS2 = 10112
RPT2 = ROWS2 // NS
OPT = 624
TAIL = N - NS * OPT
DW = 16


@functools.cache
def _sc_kernels():
    mesh = plsc.VectorSubcoreMesh(core_axis_name="c", subcore_axis_name="s",
                                  num_cores=NC, num_subcores=NS)

    @functools.partial(
        pl.kernel,
        out_type=jax.ShapeDtypeStruct((NC, N, DW), jnp.float32),
        mesh=mesh,
        scratch_types=[
            pltpu.VMEM((CHUNKS, C), jnp.int32),
            pltpu.VMEM((C, DW), jnp.float32),
            pltpu.VMEM_SHARED((ROWS, DW), jnp.float32),
            pltpu.SemaphoreType.DMA,
        ],
        compiler_params=pltpu.CompilerParams(use_tc_tiling_on_sc=False),
    )
    def sc_deg(dst_hbm, ones_hbm, zeros_hbm, out_hbm, dst_v, ones_v, acc, sem):
        cid = lax.axis_index("c")
        sid = lax.axis_index("s")
        wid = sid * NC + cid
        pltpu.sync_copy(zeros_hbm, acc.at[pl.ds(sid * RPT, RPT)])
        pltpu.sync_copy(ones_hbm, ones_v)
        pltpu.sync_copy(dst_hbm.at[pl.ds(wid * CHUNKS, CHUNKS)], dst_v)
        plsc.subcore_barrier()

        def body(t, carry):
            for k in range(4):
                pltpu.async_copy(ones_v, acc.at[dst_v.at[t * 4 + k]], sem,
                                 add=True)
            for k in range(4):
                pltpu.make_async_copy(
                    ones_v, acc.at[dst_v.at[t * 4 + k]], sem).wait()
            return carry

        lax.fori_loop(0, CHUNKS // 4, body, 0)
        plsc.subcore_barrier()
        pltpu.sync_copy(acc.at[pl.ds(sid * OPT, OPT)],
                        out_hbm.at[cid, pl.ds(sid * OPT, OPT)])

        @pl.when(sid == NS - 1)
        def _():
            pltpu.sync_copy(acc.at[pl.ds(NS * OPT, TAIL)],
                            out_hbm.at[cid, pl.ds(NS * OPT, TAIL)])

    @functools.partial(
        pl.kernel,
        out_type=jax.ShapeDtypeStruct((NC, N, HH), jnp.float32),
        mesh=mesh,
        scratch_types=(
            [pltpu.VMEM((CH2, C), jnp.int32),
             pltpu.VMEM((CH2, C), jnp.int32)]
            + [pltpu.VMEM((C, HH), jnp.float32) for _ in range(NQ * CPG)]
            + [pltpu.VMEM_SHARED((ROWS2, HH), jnp.float32),
               pltpu.VMEM_SHARED((N, HH), jnp.float32)]
            + [pltpu.SemaphoreType.DMA for _ in range(2 * NQ)]
        ),
        compiler_params=pltpu.CompilerParams(use_tc_tiling_on_sc=False),
    )
    def sc_edge(u_hbm, src_hbm, dst_hbm, zeros_hbm, out_hbm,
                src_v, dst_v, b0, b1, b2, b3, b4, b5, b6, b7, acc, u_spm,
                sg0, sg1, sg2, sg3, ss0, ss1, ss2, ss3):
        bufs = ((b0, b1), (b2, b3), (b4, b5), (b6, b7))
        sgs = (sg0, sg1, sg2, sg3)
        sss = (ss0, ss1, ss2, ss3)
        cid = lax.axis_index("c")
        sid = lax.axis_index("s")
        pltpu.sync_copy(zeros_hbm, acc.at[pl.ds(sid * RPT2, RPT2)])
        pltpu.sync_copy(src_hbm.at[pl.ds(sid * CH2, CH2)], src_v)
        pltpu.sync_copy(dst_hbm.at[pl.ds(sid * CH2, CH2)], dst_v)
        pltpu.sync_copy(u_hbm.at[cid, pl.ds(sid * OPT, OPT)],
                        u_spm.at[pl.ds(sid * OPT, OPT)])

        @pl.when(sid == NS - 1)
        def _():
            pltpu.sync_copy(u_hbm.at[cid, pl.ds(NS * OPT, TAIL)],
                            u_spm.at[pl.ds(NS * OPT, TAIL)])

        plsc.subcore_barrier()

        def fire_gather(g, q):
            for k in range(CPG):
                pltpu.async_copy(u_spm.at[src_v.at[g * CPG + k]],
                                 bufs[q][k], sgs[q])

        def drain_gather(g, q):
            for k in range(CPG):
                pltpu.make_async_copy(u_spm.at[src_v.at[g * CPG + k]],
                                      bufs[q][k], sgs[q]).wait()

        def fire_scatter(g, q):
            for k in range(CPG):
                pltpu.async_copy(bufs[q][k], acc.at[dst_v.at[g * CPG + k]],
                                 sss[q], add=True)

        def drain_scatter(g, q):
            for k in range(CPG):
                pltpu.make_async_copy(bufs[q][k],
                                      acc.at[dst_v.at[g * CPG + k]],
                                      sss[q]).wait()

        fire_gather(0, 0)
        fire_gather(1, 1)

        def step(g, q):
            @pl.when(g >= 2)
            def _():
                drain_scatter(g - 2, (q + 2) % NQ)

            @pl.when(g + 2 < NG)
            def _():
                fire_gather(g + 2, (q + 2) % NQ)

            drain_gather(g, q)
            fire_scatter(g, q)

        def body(t, carry):
            for k in range(NQ):
                step(t * NQ + k, k)
            return carry

        lax.fori_loop(0, NG // NQ, body, 0)
        drain_scatter(NG - 2, (NG - 2) % NQ)
        drain_scatter(NG - 1, (NG - 1) % NQ)
        plsc.subcore_barrier()
        pltpu.sync_copy(acc.at[pl.ds(sid * OPT, OPT)],
                        out_hbm.at[cid, pl.ds(sid * OPT, OPT)])

        @pl.when(sid == NS - 1)
        def _():
            pltpu.sync_copy(acc.at[pl.ds(NS * OPT, TAIL)],
                            out_hbm.at[cid, pl.ds(NS * OPT, TAIL)])

    return sc_deg, sc_edge


BLK = 1000


def _tc1_body(x_ref, w_ref, degp_ref, u_ref, dinv_ref):
    d = degp_ref[...]
    deg = d[0, :, :1] + d[1, :, :1] + 1.0
    dinv = lax.rsqrt(deg)
    h = jnp.dot(x_ref[...], w_ref[...],
                preferred_element_type=jnp.float32) * dinv
    u_ref[0] = h[:, :HH]
    u_ref[1] = h[:, HH:]
    dinv_ref[...] = jnp.broadcast_to(dinv, (BLK, H))


_tc1 = pl.pallas_call(
    _tc1_body,
    grid=(N // BLK,),
    in_specs=[
        pl.BlockSpec((BLK, D_IN), lambda i: (i, 0)),
        pl.BlockSpec((D_IN, H), lambda i: (0, 0)),
        pl.BlockSpec((NC, BLK, DW), lambda i: (0, i, 0)),
    ],
    out_specs=[
        pl.BlockSpec((NC, BLK, HH), lambda i: (0, i, 0)),
        pl.BlockSpec((BLK, H), lambda i: (i, 0)),
    ],
    out_shape=[
        jax.ShapeDtypeStruct((NC, N, HH), jnp.float32),
        jax.ShapeDtypeStruct((N, H), jnp.float32),
    ],
)


def _tc_mid_body(sp_ref, u_ref, dinv_ref, b_ref, w_ref, out_ref):
    sp = sp_ref[...]
    up = u_ref[...]
    s = jnp.concatenate([sp[0] + up[0], sp[1] + up[1]], axis=1)
    dinv = dinv_ref[...]
    h = jnp.maximum(s * dinv + b_ref[...], 0.0)
    m = jnp.dot(h, w_ref[...], preferred_element_type=jnp.float32) * dinv
    out_ref[0] = m[:, :HH]
    out_ref[1] = m[:, HH:]


_tc_mid = pl.pallas_call(
    _tc_mid_body,
    grid=(N // BLK,),
    in_specs=[
        pl.BlockSpec((NC, BLK, HH), lambda i: (0, i, 0)),
        pl.BlockSpec((NC, BLK, HH), lambda i: (0, i, 0)),
        pl.BlockSpec((BLK, H), lambda i: (i, 0)),
        pl.BlockSpec((1, H), lambda i: (0, 0)),
        pl.BlockSpec((H, H), lambda i: (0, 0)),
    ],
    out_specs=pl.BlockSpec((NC, BLK, HH), lambda i: (0, i, 0)),
    out_shape=jax.ShapeDtypeStruct((NC, N, HH), jnp.float32),
)


def _tc4_body(sp_ref, u_ref, dinv_ref, b_ref, batch_ref,
              wc1_ref, bc1_ref, wc2_ref, bc2_ref, out_ref):
    sp = sp_ref[...]
    up = u_ref[...]
    s = jnp.concatenate([sp[0] + up[0], sp[1] + up[1]], axis=1)
    h3 = s * dinv_ref[...] + b_ref[...]
    bt = batch_ref[...]
    sums, maxs, cnts = [], [], []
    neg_inf = jnp.float32(-jnp.inf)
    for g in range(G):
        m = bt == g
        sums.append(jnp.sum(jnp.where(m, h3, 0.0), axis=0, keepdims=True))
        maxs.append(jnp.max(jnp.where(m, h3, neg_inf), axis=0, keepdims=True))
        cnts.append(jnp.sum(m.astype(jnp.float32), axis=0, keepdims=True))
    seg_sum = jnp.concatenate(sums, axis=0)
    seg_max = jnp.concatenate(maxs, axis=0)
    counts = jnp.concatenate(cnts, axis=0)
    mean = seg_sum / jnp.maximum(counts, 1.0)
    feat = jnp.concatenate([mean, seg_max], axis=1)
    z = jnp.maximum(
        jnp.dot(feat, wc1_ref[...], preferred_element_type=jnp.float32)
        + bc1_ref[...], 0.0)
    out_ref[...] = jnp.dot(
        z, wc2_ref[...], preferred_element_type=jnp.float32) + bc2_ref[...]


_tc4 = pl.pallas_call(
    _tc4_body,
    out_shape=jax.ShapeDtypeStruct((G, OUT), jnp.float32),
)


def kernel(x, edge_index, batch, W1, b1, W2, b2, W3, b3, Wc1, bc1, Wc2, bc2):
    src = edge_index[0]
    dst = edge_index[1]
    pad = E_PAD - E
    src_p = jnp.concatenate(
        [src, jnp.zeros((pad,), jnp.int32)]).reshape(NW * CHUNKS, C)
    dst_p = jnp.concatenate(
        [dst, jnp.full((pad,), N, jnp.int32)]).reshape(NW * CHUNKS, C)
    zeros_h = jnp.zeros((RPT2, HH), jnp.float32)
    zeros_d = jnp.zeros((RPT, DW), jnp.float32)
    ones_d = jnp.ones((C, DW), jnp.float32)

    sc_deg, sc_edge = _sc_kernels()
    degp = sc_deg(dst_p, ones_d, zeros_d)
    u1, dinv = _tc1(x, W1, degp)
    s1 = sc_edge(u1, src_p, dst_p, zeros_h)
    u2 = _tc_mid(s1, u1, dinv, b1.reshape(1, H), W2)
    s2 = sc_edge(u2, src_p, dst_p, zeros_h)
    u3 = _tc_mid(s2, u2, dinv, b2.reshape(1, H), W3)
    s3 = sc_edge(u3, src_p, dst_p, zeros_h)
    logits = _tc4(s3, u3, dinv, b3.reshape(1, H), batch.reshape(N, 1),
                  Wc1, bc1.reshape(1, H), Wc2, bc2.reshape(1, OUT))
    return logits


# speedup vs baseline: 31.3812x; 1.0317x over previous
import functools

import jax
import jax.numpy as jnp
from jax import lax
from jax.experimental import pallas as pl
from jax.experimental.pallas import tpu as pltpu
from jax.experimental.pallas import tpu_sc as plsc

N = 10000
E = 320000
D_IN = 128
H = 64
G = 16
OUT = 2

NC = 2
NS = 16
NW = NC * NS
HH = H // NC

C = 128
CHUNKS = 80
EPW = C * CHUNKS
E_PAD = NW * EPW
CH2 = 160
NG = CH2 // 2
CPG = 2
NQ = 4

ROWS = 10240
RPT = ROWS // NS
ROWS2 = 10112
RPT2 = ROWS2 // NS
OPT = 624
TAIL = N - NS * OPT
DW = 16


@functools.cache
def _sc_kernels():
    mesh = plsc.VectorSubcoreMesh(core_axis_name="c", subcore_axis_name="s",
                                  num_cores=NC, num_subcores=NS)

    @functools.partial(
        pl.kernel,
        out_type=jax.ShapeDtypeStruct((NC, N, DW), jnp.float32),
        mesh=mesh,
        scratch_types=[
            pltpu.VMEM((CHUNKS, C), jnp.int32),
            pltpu.VMEM((C, DW), jnp.float32),
            pltpu.VMEM_SHARED((ROWS, DW), jnp.float32),
            pltpu.SemaphoreType.DMA,
        ],
        compiler_params=pltpu.CompilerParams(use_tc_tiling_on_sc=False),
    )
    def sc_deg(dst_hbm, ones_hbm, zeros_hbm, out_hbm, dst_v, ones_v, acc, sem):
        cid = lax.axis_index("c")
        sid = lax.axis_index("s")
        wid = sid * NC + cid
        pltpu.sync_copy(zeros_hbm, acc.at[pl.ds(sid * RPT, RPT)])
        pltpu.sync_copy(ones_hbm, ones_v)
        pltpu.sync_copy(dst_hbm.at[pl.ds(wid * CHUNKS, CHUNKS)], dst_v)
        plsc.subcore_barrier()

        def body(t, carry):
            for k in range(4):
                pltpu.async_copy(ones_v, acc.at[dst_v.at[t * 4 + k]], sem,
                                 add=True)
            for k in range(4):
                pltpu.make_async_copy(
                    ones_v, acc.at[dst_v.at[t * 4 + k]], sem).wait()
            return carry

        lax.fori_loop(0, CHUNKS // 4, body, 0)
        plsc.subcore_barrier()
        pltpu.sync_copy(acc.at[pl.ds(sid * OPT, OPT)],
                        out_hbm.at[cid, pl.ds(sid * OPT, OPT)])

        @pl.when(sid == NS - 1)
        def _():
            pltpu.sync_copy(acc.at[pl.ds(NS * OPT, TAIL)],
                            out_hbm.at[cid, pl.ds(NS * OPT, TAIL)])

    @functools.partial(
        pl.kernel,
        out_type=jax.ShapeDtypeStruct((NC, N, HH), jnp.float32),
        mesh=mesh,
        scratch_types=(
            [pltpu.VMEM((CH2, C), jnp.int32),
             pltpu.VMEM((CH2, C), jnp.int32)]
            + [pltpu.VMEM((C, HH), jnp.float32) for _ in range(NQ * CPG)]
            + [pltpu.VMEM_SHARED((ROWS2, HH), jnp.float32),
               pltpu.VMEM_SHARED((N, HH), jnp.float32)]
            + [pltpu.SemaphoreType.DMA for _ in range(2 * NQ)]
        ),
        compiler_params=pltpu.CompilerParams(use_tc_tiling_on_sc=False),
    )
    def sc_edge(u_hbm, src_hbm, dst_hbm, zeros_hbm, out_hbm,
                src_v, dst_v, b0, b1, b2, b3, b4, b5, b6, b7, acc, u_spm,
                sg0, sg1, sg2, sg3, ss0, ss1, ss2, ss3):
        bufs = ((b0, b1), (b2, b3), (b4, b5), (b6, b7))
        sgs = (sg0, sg1, sg2, sg3)
        sss = (ss0, ss1, ss2, ss3)
        cid = lax.axis_index("c")
        sid = lax.axis_index("s")
        pltpu.sync_copy(zeros_hbm, acc.at[pl.ds(sid * RPT2, RPT2)])
        pltpu.sync_copy(src_hbm.at[pl.ds(sid * CH2, CH2)], src_v)
        pltpu.sync_copy(dst_hbm.at[pl.ds(sid * CH2, CH2)], dst_v)
        pltpu.sync_copy(u_hbm.at[cid, pl.ds(sid * OPT, OPT)],
                        u_spm.at[pl.ds(sid * OPT, OPT)])

        @pl.when(sid == NS - 1)
        def _():
            pltpu.sync_copy(u_hbm.at[cid, pl.ds(NS * OPT, TAIL)],
                            u_spm.at[pl.ds(NS * OPT, TAIL)])

        plsc.subcore_barrier()

        def fire_gather(g, q):
            for k in range(CPG):
                pltpu.async_copy(u_spm.at[src_v.at[g * CPG + k]],
                                 bufs[q][k], sgs[q])

        def drain_gather(g, q):
            for k in range(CPG):
                pltpu.make_async_copy(u_spm.at[src_v.at[g * CPG + k]],
                                      bufs[q][k], sgs[q]).wait()

        def fire_scatter(g, q):
            for k in range(CPG):
                pltpu.async_copy(bufs[q][k], acc.at[dst_v.at[g * CPG + k]],
                                 sss[q], add=True)

        def drain_scatter(g, q):
            for k in range(CPG):
                pltpu.make_async_copy(bufs[q][k],
                                      acc.at[dst_v.at[g * CPG + k]],
                                      sss[q]).wait()

        fire_gather(0, 0)
        fire_gather(1, 1)

        def step(g, q):
            @pl.when(g >= 2)
            def _():
                drain_scatter(g - 2, (q + 2) % NQ)

            @pl.when(g + 2 < NG)
            def _():
                fire_gather(g + 2, (q + 2) % NQ)

            drain_gather(g, q)
            fire_scatter(g, q)

        def body(t, carry):
            for k in range(NQ):
                step(t * NQ + k, k)
            return carry

        lax.fori_loop(0, NG // NQ, body, 0)
        drain_scatter(NG - 2, (NG - 2) % NQ)
        drain_scatter(NG - 1, (NG - 1) % NQ)
        plsc.subcore_barrier()
        pltpu.sync_copy(acc.at[pl.ds(sid * OPT, OPT)],
                        out_hbm.at[cid, pl.ds(sid * OPT, OPT)])

        @pl.when(sid == NS - 1)
        def _():
            pltpu.sync_copy(acc.at[pl.ds(NS * OPT, TAIL)],
                            out_hbm.at[cid, pl.ds(NS * OPT, TAIL)])

    return sc_deg, sc_edge


BLK = 1000


def _tc1_body(x_ref, w_ref, degp_ref, u_ref, dinv_ref):
    d = degp_ref[...]
    deg = d[0, :, :1] + d[1, :, :1] + 1.0
    dinv = lax.rsqrt(deg)
    h = jnp.dot(x_ref[...], w_ref[...],
                preferred_element_type=jnp.float32) * dinv
    u_ref[0] = h[:, :HH]
    u_ref[1] = h[:, HH:]
    dinv_ref[...] = jnp.broadcast_to(dinv, (BLK, H))


_tc1 = pl.pallas_call(
    _tc1_body,
    grid=(N // BLK,),
    in_specs=[
        pl.BlockSpec((BLK, D_IN), lambda i: (i, 0)),
        pl.BlockSpec((D_IN, H), lambda i: (0, 0)),
        pl.BlockSpec((NC, BLK, DW), lambda i: (0, i, 0)),
    ],
    out_specs=[
        pl.BlockSpec((NC, BLK, HH), lambda i: (0, i, 0)),
        pl.BlockSpec((BLK, H), lambda i: (i, 0)),
    ],
    out_shape=[
        jax.ShapeDtypeStruct((NC, N, HH), jnp.float32),
        jax.ShapeDtypeStruct((N, H), jnp.float32),
    ],
)


def _tc_mid_body(sp_ref, u_ref, dinv_ref, b_ref, w_ref, out_ref):
    sp = sp_ref[...]
    up = u_ref[...]
    s = jnp.concatenate([sp[0] + up[0], sp[1] + up[1]], axis=1)
    dinv = dinv_ref[...]
    h = jnp.maximum(s * dinv + b_ref[...], 0.0)
    m = jnp.dot(h, w_ref[...], preferred_element_type=jnp.float32) * dinv
    out_ref[0] = m[:, :HH]
    out_ref[1] = m[:, HH:]


_tc_mid = pl.pallas_call(
    _tc_mid_body,
    grid=(N // BLK,),
    in_specs=[
        pl.BlockSpec((NC, BLK, HH), lambda i: (0, i, 0)),
        pl.BlockSpec((NC, BLK, HH), lambda i: (0, i, 0)),
        pl.BlockSpec((BLK, H), lambda i: (i, 0)),
        pl.BlockSpec((1, H), lambda i: (0, 0)),
        pl.BlockSpec((H, H), lambda i: (0, 0)),
    ],
    out_specs=pl.BlockSpec((NC, BLK, HH), lambda i: (0, i, 0)),
    out_shape=jax.ShapeDtypeStruct((NC, N, HH), jnp.float32),
)


def _tc4_body(sp_ref, u_ref, dinv_ref, b_ref, batch_ref, batch_row_ref,
              wc1_ref, bc1_ref, wc2_ref, bc2_ref, out_ref):
    sp = sp_ref[...]
    up = u_ref[...]
    s = jnp.concatenate([sp[0] + up[0], sp[1] + up[1]], axis=1)
    h3 = s * dinv_ref[...] + b_ref[...]
    bt = batch_ref[...]
    btr = batch_row_ref[...]
    oneh = (lax.broadcasted_iota(jnp.int32, (G, 1), 0) == btr
            ).astype(jnp.float32)
    seg_sum = jnp.dot(oneh, h3, preferred_element_type=jnp.float32)
    counts = jnp.sum(oneh, axis=1, keepdims=True)
    maxs = []
    neg_inf = jnp.float32(-jnp.inf)
    for g in range(G):
        m = bt == g
        maxs.append(jnp.max(jnp.where(m, h3, neg_inf), axis=0, keepdims=True))
    seg_max = jnp.concatenate(maxs, axis=0)
    mean = seg_sum / jnp.maximum(counts, 1.0)
    feat = jnp.concatenate([mean, seg_max], axis=1)
    z = jnp.maximum(
        jnp.dot(feat, wc1_ref[...], preferred_element_type=jnp.float32)
        + bc1_ref[...], 0.0)
    out_ref[...] = jnp.dot(
        z, wc2_ref[...], preferred_element_type=jnp.float32) + bc2_ref[...]


_tc4 = pl.pallas_call(
    _tc4_body,
    out_shape=jax.ShapeDtypeStruct((G, OUT), jnp.float32),
    compiler_params=pltpu.CompilerParams(
        vmem_limit_bytes=100 * 1024 * 1024),
)


def kernel(x, edge_index, batch, W1, b1, W2, b2, W3, b3, Wc1, bc1, Wc2, bc2):
    src = edge_index[0]
    dst = edge_index[1]
    pad = E_PAD - E
    src_p = jnp.concatenate(
        [src, jnp.zeros((pad,), jnp.int32)]).reshape(NW * CHUNKS, C)
    dst_p = jnp.concatenate(
        [dst, jnp.full((pad,), N, jnp.int32)]).reshape(NW * CHUNKS, C)
    zeros_h = jnp.zeros((RPT2, HH), jnp.float32)
    zeros_d = jnp.zeros((RPT, DW), jnp.float32)
    ones_d = jnp.ones((C, DW), jnp.float32)

    sc_deg, sc_edge = _sc_kernels()
    degp = sc_deg(dst_p, ones_d, zeros_d)
    u1, dinv = _tc1(x, W1, degp)
    s1 = sc_edge(u1, src_p, dst_p, zeros_h)
    u2 = _tc_mid(s1, u1, dinv, b1.reshape(1, H), W2)
    s2 = sc_edge(u2, src_p, dst_p, zeros_h)
    u3 = _tc_mid(s2, u2, dinv, b2.reshape(1, H), W3)
    s3 = sc_edge(u3, src_p, dst_p, zeros_h)
    logits = _tc4(s3, u3, dinv, b3.reshape(1, H), batch.reshape(N, 1),
                  batch.reshape(1, N),
                  Wc1, bc1.reshape(1, H), Wc2, bc2.reshape(1, OUT))
    return logits
